# baseline (speedup 1.0000x reference)
import functools

import jax
import jax.numpy as jnp
from jax import lax
from jax.experimental import pallas as pl
from jax.experimental.pallas import tpu as pltpu
from jax.experimental.pallas import tpu_sc as plsc

N = 10000
D = 128
NC = 2
NS = 16
NW = NC * NS
CHUNK = 128
E = 320000
NCHUNKS = 80
EW = NCHUNKS * CHUNK
EP = EW * NW
NBUF = 2
NPAD = 10112
ROWS_PER_SUB = NPAD // NS

_mesh = plsc.VectorSubcoreMesh(
    core_axis_name="c", subcore_axis_name="s", num_cores=NC, num_subcores=NS
)


@functools.partial(
    pl.kernel,
    out_type=jax.ShapeDtypeStruct((NC * NPAD,), jnp.float32),
    mesh=_mesh,
    scratch_types=[
        pltpu.VMEM((NCHUNKS, CHUNK), jnp.int32),
        pltpu.VMEM((CHUNK,), jnp.float32),
        pltpu.VMEM((NPAD,), jnp.float32),
        pltpu.VMEM_SHARED((NPAD,), jnp.float32),
        pltpu.SemaphoreType.DMA,
    ],
)
def _deg_kernel(dst_hbm, ones_hbm, z_hbm, out_hbm, dst_v, ones_v, bounce, hist, sem):
    cid = lax.axis_index("c")
    sid = lax.axis_index("s")
    wid = sid * NC + cid
    base_r = sid * ROWS_PER_SUB
    pltpu.sync_copy(z_hbm, bounce)
    pltpu.sync_copy(bounce.at[pl.ds(0, ROWS_PER_SUB)], hist.at[pl.ds(base_r, ROWS_PER_SUB)])
    pltpu.sync_copy(dst_hbm.at[wid], dst_v)
    pltpu.sync_copy(ones_hbm, ones_v)
    plsc.subcore_barrier()

    def fire(c, carry):
        pltpu.async_copy(ones_v, hist.at[dst_v.at[c]], sem, add=True)
        return carry

    lax.fori_loop(0, NCHUNKS, fire, 0)

    def drain(c, carry):
        pltpu.make_async_copy(ones_v, hist.at[dst_v.at[0]], sem).wait()
        return carry

    lax.fori_loop(0, NCHUNKS, drain, 0)
    plsc.subcore_barrier()
    pltpu.sync_copy(hist.at[pl.ds(base_r, ROWS_PER_SUB)], bounce.at[pl.ds(0, ROWS_PER_SUB)])
    pltpu.sync_copy(
        bounce.at[pl.ds(0, ROWS_PER_SUB)],
        out_hbm.at[pl.ds(cid * NPAD + base_r, ROWS_PER_SUB)],
    )


@functools.partial(
    pl.kernel,
    out_type=jax.ShapeDtypeStruct((NC, NPAD, D), jnp.float32),
    mesh=_mesh,
    scratch_types=[
        [pltpu.VMEM((CHUNK,), jnp.int32)] * 3,
        [pltpu.VMEM((CHUNK,), jnp.int32)] * 3,
        pltpu.VMEM((3, CHUNK, D), jnp.float32),
        pltpu.VMEM_SHARED((NPAD, D), jnp.float32),
        [pltpu.SemaphoreType.DMA] * 3,
        [pltpu.SemaphoreType.DMA] * 3,
        [pltpu.SemaphoreType.DMA] * 3,
    ],
)
def _prop_kernel(hs_hbm, src_hbm, dst_hbm, zrow_hbm, out_hbm,
                 islots, dslots, rows, acc, isems, dsems, gsems):
    cid = lax.axis_index("c")
    sid = lax.axis_index("s")
    wid = sid * NC + cid
    base_r = sid * ROWS_PER_SUB
    for k in range(ROWS_PER_SUB // 128):
        pltpu.sync_copy(zrow_hbm, acc.at[pl.ds(base_r + k * 128, 128)])
    rem = ROWS_PER_SUB % 128
    if rem:
        pltpu.sync_copy(
            zrow_hbm.at[pl.ds(0, rem)],
            acc.at[pl.ds(base_r + (ROWS_PER_SUB // 128) * 128, rem)],
        )

    def start_idx(c, b):
        pltpu.async_copy(src_hbm.at[wid, c], islots[b], isems[b])
        pltpu.async_copy(dst_hbm.at[wid, c], dslots[b], dsems[b])

    def wait_idx_start_gather(c, b):
        pltpu.make_async_copy(src_hbm.at[wid, 0], islots[b], isems[b]).wait()
        pltpu.async_copy(hs_hbm.at[islots[b]], rows.at[b], gsems[b])

    def finish(c, b):
        pltpu.make_async_copy(hs_hbm.at[islots[b]], rows.at[b], gsems[b]).wait()
        pltpu.make_async_copy(dst_hbm.at[wid, 0], dslots[b], dsems[b]).wait()
        pltpu.sync_copy(rows.at[b], acc.at[dslots[b]], add=True)

    for k in range(3):
        start_idx(k, k)
    for k in range(2):
        wait_idx_start_gather(k, k)
    plsc.subcore_barrier()

    def group(g, carry):
        for b in range(3):
            c = g * 3 + b
            wait_idx_start_gather(c + 2, (b + 2) % 3)
            finish(c, b)
            start_idx(c + 3, b)
        return carry

    lax.fori_loop(0, (NCHUNKS - 5) // 3, group, 0)
    for c in range(NCHUNKS - 5, NCHUNKS):
        b = c % 3
        if c + 2 < NCHUNKS:
            wait_idx_start_gather(c + 2, (b + 2) % 3)
        finish(c, b)
        if c + 3 < NCHUNKS:
            start_idx(c + 3, b)
    plsc.subcore_barrier()
    pltpu.sync_copy(
        acc.at[pl.ds(base_r, ROWS_PER_SUB)],
        out_hbm.at[cid, pl.ds(base_r, ROWS_PER_SUB)],
    )


def _tc1_body(parts_ref, x_ref, w_ref, dinv_ref, hs_ref):
    deg = parts_ref[0, 0:N] + parts_ref[1, 0:N] + 1.0
    dinv = lax.rsqrt(deg)
    h = jnp.dot(x_ref[...], w_ref[...], preferred_element_type=jnp.float32)
    dinv_ref[...] = dinv
    hs_ref[...] = h * dinv


_tc1 = pl.pallas_call(
    _tc1_body,
    out_shape=(
        jax.ShapeDtypeStruct((N, 1), jnp.float32),
        jax.ShapeDtypeStruct((N, D), jnp.float32),
    ),
)


def _tc_mid_body(p_ref, hs_ref, dinv_ref, b_ref, g_ref, be_ref, w_ref, out_ref):
    dinv = dinv_ref[...]
    o = (p_ref[0, 0:N] + p_ref[1, 0:N] + hs_ref[...]) * dinv + b_ref[...]
    mean = jnp.mean(o, axis=0, keepdims=True)
    cen = o - mean
    var = jnp.mean(cen * cen, axis=0, keepdims=True)
    o = cen * lax.rsqrt(var + 1e-5) * g_ref[...] + be_ref[...]
    o = jnp.maximum(o, 0.0)
    h = jnp.dot(o, w_ref[...], preferred_element_type=jnp.float32)
    out_ref[...] = h * dinv


_tc_mid = pl.pallas_call(
    _tc_mid_body,
    out_shape=jax.ShapeDtypeStruct((N, D), jnp.float32),
)


def _tc_fin_body(p_ref, hs_ref, dinv_ref, b_ref, out_ref):
    out_ref[...] = (p_ref[0, 0:N] + p_ref[1, 0:N] + hs_ref[...]) * dinv_ref[...] + b_ref[...]


_tc_fin = pl.pallas_call(
    _tc_fin_body,
    out_shape=jax.ShapeDtypeStruct((N, D), jnp.float32),
)


def kernel(x, edge_index, W1, b1, g1, be1, W2, b2, g2, be2, W3, b3):
    src = edge_index[0]
    dst = edge_index[1]
    padn = EP - E
    fill = jnp.arange(padn, dtype=jnp.int32)
    src_p = jnp.concatenate([src, fill % N]).reshape(NW, NCHUNKS, CHUNK)
    dst_p = jnp.concatenate([dst, N + (fill % (NPAD - N))]).reshape(NW, NCHUNKS, CHUNK)

    zrow = jnp.zeros((128, D), jnp.float32)
    zcol = jnp.zeros((NPAD,), jnp.float32)
    ones1 = jnp.ones((CHUNK,), jnp.float32)

    b1r, g1r, be1r = b1.reshape(1, D), g1.reshape(1, D), be1.reshape(1, D)
    b2r, g2r, be2r = b2.reshape(1, D), g2.reshape(1, D), be2.reshape(1, D)
    b3r = b3.reshape(1, D)

    deg_parts = _deg_kernel(dst_p, ones1, zcol).reshape(NC, NPAD, 1)
    dinv, hs1 = _tc1(deg_parts, x, W1)
    p1 = _prop_kernel(hs1, src_p, dst_p, zrow)
    hs2 = _tc_mid(p1, hs1, dinv, b1r, g1r, be1r, W2)
    p2 = _prop_kernel(hs2, src_p, dst_p, zrow)
    hs3 = _tc_mid(p2, hs2, dinv, b2r, g2r, be2r, W3)
    p3 = _prop_kernel(hs3, src_p, dst_p, zrow)
    out = _tc_fin(p3, hs3, dinv, b3r)
    return out


# speedup vs baseline: 1.1235x; 1.1235x over previous
import functools

import jax
import jax.numpy as jnp
from jax import lax
from jax.experimental import pallas as pl
from jax.experimental.pallas import tpu as pltpu
from jax.experimental.pallas import tpu_sc as plsc

N = 10000
D = 128
NC = 2
NS = 16
NW = NC * NS
CHUNK = 128
E = 320000
NCHUNKS = 80
EW = NCHUNKS * CHUNK
EP = EW * NW
NBUF = 2
NPAD = 10112
ROWS_PER_SUB = NPAD // NS

_mesh = plsc.VectorSubcoreMesh(
    core_axis_name="c", subcore_axis_name="s", num_cores=NC, num_subcores=NS
)


@functools.partial(
    pl.kernel,
    out_type=jax.ShapeDtypeStruct((NC * NPAD,), jnp.float32),
    mesh=_mesh,
    scratch_types=[
        pltpu.VMEM((NCHUNKS, CHUNK), jnp.int32),
        pltpu.VMEM((CHUNK,), jnp.float32),
        pltpu.VMEM((NPAD,), jnp.float32),
        pltpu.VMEM_SHARED((NPAD,), jnp.float32),
        pltpu.SemaphoreType.DMA,
    ],
)
def _deg_kernel(dst_hbm, ones_hbm, z_hbm, out_hbm, dst_v, ones_v, bounce, hist, sem):
    cid = lax.axis_index("c")
    sid = lax.axis_index("s")
    wid = sid * NC + cid
    base_r = sid * ROWS_PER_SUB
    pltpu.sync_copy(z_hbm, bounce)
    pltpu.sync_copy(bounce.at[pl.ds(0, ROWS_PER_SUB)], hist.at[pl.ds(base_r, ROWS_PER_SUB)])
    pltpu.sync_copy(dst_hbm.at[wid], dst_v)
    pltpu.sync_copy(ones_hbm, ones_v)
    plsc.subcore_barrier()

    def fire(c, carry):
        pltpu.async_copy(ones_v, hist.at[dst_v.at[c]], sem, add=True)
        return carry

    lax.fori_loop(0, NCHUNKS, fire, 0)

    def drain(c, carry):
        pltpu.make_async_copy(ones_v, hist.at[dst_v.at[0]], sem).wait()
        return carry

    lax.fori_loop(0, NCHUNKS, drain, 0)
    plsc.subcore_barrier()
    pltpu.sync_copy(hist.at[pl.ds(base_r, ROWS_PER_SUB)], bounce.at[pl.ds(0, ROWS_PER_SUB)])
    pltpu.sync_copy(
        bounce.at[pl.ds(0, ROWS_PER_SUB)],
        out_hbm.at[pl.ds(cid * NPAD + base_r, ROWS_PER_SUB)],
    )


@functools.partial(
    pl.kernel,
    out_type=jax.ShapeDtypeStruct((NC, NPAD, D), jnp.float32),
    mesh=_mesh,
    scratch_types=[
        [pltpu.VMEM((CHUNK,), jnp.int32)] * 3,
        [pltpu.VMEM((CHUNK,), jnp.int32)] * 3,
        pltpu.VMEM((3, CHUNK, D), jnp.float32),
        pltpu.VMEM_SHARED((NPAD, D), jnp.float32),
        [pltpu.SemaphoreType.DMA] * 3,
        [pltpu.SemaphoreType.DMA] * 3,
        [pltpu.SemaphoreType.DMA] * 3,
        [pltpu.SemaphoreType.DMA] * 3,
    ],
)
def _prop_kernel(hs_hbm, src_hbm, dst_hbm, zrow_hbm, out_hbm,
                 islots, dslots, rows, acc, isems, dsems, gsems, ssems):
    cid = lax.axis_index("c")
    sid = lax.axis_index("s")
    wid = sid * NC + cid
    base_r = sid * ROWS_PER_SUB
    for k in range(ROWS_PER_SUB // 128):
        pltpu.sync_copy(zrow_hbm, acc.at[pl.ds(base_r + k * 128, 128)])
    rem = ROWS_PER_SUB % 128
    if rem:
        pltpu.sync_copy(
            zrow_hbm.at[pl.ds(0, rem)],
            acc.at[pl.ds(base_r + (ROWS_PER_SUB // 128) * 128, rem)],
        )

    def load_src(c, b):
        pltpu.async_copy(src_hbm.at[wid, c], islots[b], isems[b])

    def load_dst(c, b):
        pltpu.async_copy(dst_hbm.at[wid, c], dslots[b], dsems[b])

    def wait_src(b):
        pltpu.make_async_copy(src_hbm.at[wid, 0], islots[b], isems[b]).wait()

    def wait_dst(b):
        pltpu.make_async_copy(dst_hbm.at[wid, 0], dslots[b], dsems[b]).wait()

    def start_gather(b):
        pltpu.async_copy(hs_hbm.at[islots[b]], rows.at[b], gsems[b])

    def wait_gather(b):
        pltpu.make_async_copy(hs_hbm.at[islots[b]], rows.at[b], gsems[b]).wait()

    def fire_scatter(b):
        pltpu.async_copy(rows.at[b], acc.at[dslots[b]], ssems[b], add=True)

    def wait_scatter(b):
        pltpu.make_async_copy(rows.at[b], acc.at[dslots[b]], ssems[b]).wait()

    def visit(v, b, first=False, has_g2=True, has_s3=True):
        b2 = (b + 2) % 3
        wait_gather(b)
        wait_dst(b)
        fire_scatter(b)
        if not first:
            wait_scatter(b2)
        if has_g2:
            wait_src(b2)
            start_gather(b2)
            load_dst(v + 2, b2)
        if has_s3:
            load_src(v + 3, b)

    for k in range(3):
        load_src(k, k)
    for k in range(2):
        load_dst(k, k)
    for k in range(2):
        wait_src(k)
        start_gather(k)
    plsc.subcore_barrier()

    visit(0, 0, first=True)

    def group(g, carry):
        for j in range(3):
            visit(g * 3 + 1 + j, (1 + j) % 3)
        return carry

    lax.fori_loop(0, (NCHUNKS - 4) // 3, group, 0)
    for v in range(NCHUNKS - 4, NCHUNKS):
        visit(v, v % 3, has_g2=(v + 2 < NCHUNKS), has_s3=(v + 3 < NCHUNKS))
    wait_scatter((NCHUNKS - 1) % 3)
    plsc.subcore_barrier()
    pltpu.sync_copy(
        acc.at[pl.ds(base_r, ROWS_PER_SUB)],
        out_hbm.at[cid, pl.ds(base_r, ROWS_PER_SUB)],
    )


def _tc1_body(parts_ref, x_ref, w_ref, dinv_ref, hs_ref):
    deg = parts_ref[0, 0:N] + parts_ref[1, 0:N] + 1.0
    dinv = lax.rsqrt(deg)
    h = jnp.dot(x_ref[...], w_ref[...], preferred_element_type=jnp.float32)
    dinv_ref[...] = dinv
    hs_ref[...] = h * dinv


_tc1 = pl.pallas_call(
    _tc1_body,
    out_shape=(
        jax.ShapeDtypeStruct((N, 1), jnp.float32),
        jax.ShapeDtypeStruct((N, D), jnp.float32),
    ),
)


def _tc_mid_body(p_ref, hs_ref, dinv_ref, b_ref, g_ref, be_ref, w_ref, out_ref):
    dinv = dinv_ref[...]
    o = (p_ref[0, 0:N] + p_ref[1, 0:N] + hs_ref[...]) * dinv + b_ref[...]
    mean = jnp.mean(o, axis=0, keepdims=True)
    cen = o - mean
    var = jnp.mean(cen * cen, axis=0, keepdims=True)
    o = cen * lax.rsqrt(var + 1e-5) * g_ref[...] + be_ref[...]
    o = jnp.maximum(o, 0.0)
    h = jnp.dot(o, w_ref[...], preferred_element_type=jnp.float32)
    out_ref[...] = h * dinv


_tc_mid = pl.pallas_call(
    _tc_mid_body,
    out_shape=jax.ShapeDtypeStruct((N, D), jnp.float32),
)


def _tc_fin_body(p_ref, hs_ref, dinv_ref, b_ref, out_ref):
    out_ref[...] = (p_ref[0, 0:N] + p_ref[1, 0:N] + hs_ref[...]) * dinv_ref[...] + b_ref[...]


_tc_fin = pl.pallas_call(
    _tc_fin_body,
    out_shape=jax.ShapeDtypeStruct((N, D), jnp.float32),
)


def kernel(x, edge_index, W1, b1, g1, be1, W2, b2, g2, be2, W3, b3):
    src = edge_index[0]
    dst = edge_index[1]
    padn = EP - E
    fill = jnp.arange(padn, dtype=jnp.int32)
    src_p = jnp.concatenate([src, fill % N]).reshape(NW, NCHUNKS, CHUNK)
    dst_p = jnp.concatenate([dst, N + (fill % (NPAD - N))]).reshape(NW, NCHUNKS, CHUNK)

    zrow = jnp.zeros((128, D), jnp.float32)
    zcol = jnp.zeros((NPAD,), jnp.float32)
    ones1 = jnp.ones((CHUNK,), jnp.float32)

    b1r, g1r, be1r = b1.reshape(1, D), g1.reshape(1, D), be1.reshape(1, D)
    b2r, g2r, be2r = b2.reshape(1, D), g2.reshape(1, D), be2.reshape(1, D)
    b3r = b3.reshape(1, D)

    deg_parts = _deg_kernel(dst_p, ones1, zcol).reshape(NC, NPAD, 1)
    dinv, hs1 = _tc1(deg_parts, x, W1)
    p1 = _prop_kernel(hs1, src_p, dst_p, zrow)
    hs2 = _tc_mid(p1, hs1, dinv, b1r, g1r, be1r, W2)
    p2 = _prop_kernel(hs2, src_p, dst_p, zrow)
    hs3 = _tc_mid(p2, hs2, dinv, b2r, g2r, be2r, W3)
    p3 = _prop_kernel(hs3, src_p, dst_p, zrow)
    out = _tc_fin(p3, hs3, dinv, b3r)
    return out
